# baseline (speedup 1.0000x reference)
import functools

import jax
import jax.numpy as jnp
from jax import lax
from jax.experimental import pallas as pl
from jax.experimental.pallas import tpu as pltpu
from jax.experimental.pallas import tpu_sc as plsc

NC, NS = 2, 16
NW = NC * NS
CH = 80
HQ = 8
CS = 125
WIN = 16
B = 1024


def _hist_call(NP, E):
    EPW = E // NW
    NCH = EPW // CH
    mesh = plsc.VectorSubcoreMesh(core_axis_name="c", subcore_axis_name="s")

    ZB = NP // NS

    @functools.partial(
        pl.kernel,
        out_type=jax.ShapeDtypeStruct((NC, NP), jnp.float32),
        mesh=mesh,
        scratch_types=[
            pltpu.VMEM((NCH, CH), jnp.int32),
            pltpu.VMEM((CH,), jnp.float32),
            pltpu.VMEM((ZB,), jnp.float32),
            pltpu.VMEM_SHARED((NP,), jnp.float32),
            pltpu.SemaphoreType.DMA,
        ],
    )
    def hist(dst3, out, idx_v, ones_v, zb, hist_sh, sem):
        cid = lax.axis_index("c")
        sid = lax.axis_index("s")
        wid = sid * NC + cid
        pltpu.sync_copy(dst3.at[wid], idx_v)
        for k in range(CH // 16):
            ones_v[pl.ds(k * 16, 16)] = jnp.ones((16,), jnp.float32)
        for k in range(ZB // 16):
            zb[pl.ds(k * 16, 16)] = jnp.zeros((16,), jnp.float32)
        pltpu.sync_copy(zb, hist_sh.at[pl.ds(sid * ZB, ZB)])
        plsc.subcore_barrier()

        def fire(j):
            pltpu.async_copy(ones_v, hist_sh.at[idx_v.at[j]], sem, add=True)

        def drain():
            pltpu.make_async_copy(ones_v, hist_sh.at[idx_v.at[0]], sem).wait()

        for j in range(HQ):
            fire(j)

        def body(j, carry):
            drain()
            fire(j + HQ)
            return carry

        lax.fori_loop(0, NCH - HQ, body, 0)
        for _ in range(HQ):
            drain()
        plsc.subcore_barrier()

        @pl.when(sid == 0)
        def _():
            pltpu.sync_copy(hist_sh, out.at[cid])

    return hist


def _scatter_call(N, E, D):
    EPW = E // NW
    NCH = EPW // CS
    NWIN = NCH // WIN
    RPT = 632
    RLAST = N - (NS - 1) * RPT
    mesh = plsc.VectorSubcoreMesh(core_axis_name="c", subcore_axis_name="s")

    @functools.partial(
        pl.kernel,
        out_type=[
            jax.ShapeDtypeStruct((N, D), jnp.float32),
            jax.ShapeDtypeStruct((N, D), jnp.float32),
        ],
        mesh=mesh,
        scratch_types=[
            pltpu.VMEM((WIN, CS), jnp.int32),
            pltpu.VMEM((WIN, CS), jnp.int32),
            pltpu.VMEM((WIN, CS), jnp.int32),
            pltpu.VMEM((WIN, CS), jnp.int32),
            pltpu.VMEM((2, CS, D), jnp.float32),
            pltpu.VMEM_SHARED((N, D), jnp.float32),
            pltpu.SemaphoreType.DMA,
            pltpu.SemaphoreType.DMA,
            pltpu.SemaphoreType.DMA,
            pltpu.SemaphoreType.DMA,
        ],
    )
    def scat(y, src4, dst4, out0, out1,
             iws0, iws1, iwd0, iwd1, rows, acc, g0, g1, i0, i1):
        cid = lax.axis_index("c")
        sid = lax.axis_index("s")
        wid = sid * NC + cid
        wbase = wid * NWIN
        r0 = pl.multiple_of(sid * RPT, 8)

        wbuf = ((iws0, iwd0), (iws1, iwd1))
        isem = (i0, i1)
        gsem = (g0, g1)

        def start_window(w, p):
            pltpu.async_copy(src4.at[wbase + w], wbuf[p][0], isem[p])
            pltpu.async_copy(dst4.at[wbase + w], wbuf[p][1], isem[p])

        def wait_window(p):
            pltpu.make_async_copy(src4.at[0], wbuf[p][0], isem[p]).wait()
            pltpu.make_async_copy(src4.at[0], wbuf[p][1], isem[p]).wait()

        def start_gather(idx_row, buf):
            pltpu.async_copy(y.at[idx_row], rows.at[buf], gsem[buf])

        def wait_gather(buf):
            pltpu.make_async_copy(y.at[iws0.at[0]], rows.at[buf], gsem[buf]).wait()

        def scatter(idx_row, buf):
            pltpu.sync_copy(rows.at[buf], acc.at[idx_row], add=True)

        start_window(0, 0)
        start_window(1, 1)
        wait_window(0)
        start_gather(iws0.at[0], 0)
        start_gather(iws0.at[1], 1)

        @pl.when(sid < NS - 1)
        def _():
            pltpu.sync_copy(y.at[pl.ds(r0, RPT)], acc.at[pl.ds(r0, RPT)])

        @pl.when(sid == NS - 1)
        def _():
            pltpu.sync_copy(y.at[pl.ds(r0, RLAST)], acc.at[pl.ds(r0, RLAST)])

        plsc.subcore_barrier()

        for w in range(NWIN):
            p = w % 2
            iws, iwd = wbuf[p]

            def pair(k2, carry, iws=iws, iwd=iwd):
                k = 2 * k2
                wait_gather(0)
                scatter(iwd.at[k], 0)
                start_gather(iws.at[k + 2], 0)
                wait_gather(1)
                scatter(iwd.at[k + 1], 1)
                start_gather(iws.at[k + 3], 1)
                return carry

            lax.fori_loop(0, (WIN - 2) // 2, pair, 0)
            wait_gather(0)
            scatter(iwd.at[WIN - 2], 0)
            if w + 1 < NWIN:
                wait_window(1 - p)
                start_gather(wbuf[1 - p][0].at[0], 0)
            wait_gather(1)
            scatter(iwd.at[WIN - 1], 1)
            if w + 1 < NWIN:
                start_gather(wbuf[1 - p][0].at[1], 1)
                if w + 2 < NWIN:
                    start_window(w + 2, p)

        plsc.subcore_barrier()

        def copy_out(dst_ref):
            @pl.when(sid < NS - 1)
            def _():
                pltpu.sync_copy(acc.at[pl.ds(r0, RPT)], dst_ref.at[pl.ds(r0, RPT)])

            @pl.when(sid == NS - 1)
            def _():
                pltpu.sync_copy(acc.at[pl.ds(r0, RLAST)], dst_ref.at[pl.ds(r0, RLAST)])

        @pl.when(cid == 0)
        def _():
            copy_out(out0)

        @pl.when(cid == 1)
        def _():
            copy_out(out1)

    return scat


def _matmul_call(N, D):
    def body(x_ref, w_ref, xw_ref):
        xw_ref[...] = jnp.dot(x_ref[...], w_ref[...],
                              preferred_element_type=jnp.float32)

    return pl.pallas_call(
        body,
        grid=(pl.cdiv(N, B),),
        in_specs=[
            pl.BlockSpec((B, D), lambda i: (i, 0)),
            pl.BlockSpec((D, D), lambda i: (0, 0)),
        ],
        out_specs=pl.BlockSpec((B, D), lambda i: (i, 0)),
        out_shape=jax.ShapeDtypeStruct((N, D), jnp.float32),
    )


def _scale_call(N, NP, D):
    def body(hist_ref, xw_ref, y_ref):
        i = pl.program_id(0)
        off = pl.multiple_of(i * B, 128)
        h = hist_ref[0, pl.ds(off, B)] + hist_ref[1, pl.ds(off, B)]
        dis = lax.rsqrt(1.0 + h)
        y_ref[...] = xw_ref[...] * dis[:, None]

    return pl.pallas_call(
        body,
        grid=(pl.cdiv(N, B),),
        in_specs=[
            pl.BlockSpec((NC, NP), lambda i: (0, 0)),
            pl.BlockSpec((B, D), lambda i: (i, 0)),
        ],
        out_specs=pl.BlockSpec((B, D), lambda i: (i, 0)),
        out_shape=jax.ShapeDtypeStruct((N, D), jnp.float32),
    )


def _final_call(N, NP, D):
    def body(hist_ref, a0_ref, a1_ref, y_ref, b_ref, o_ref):
        i = pl.program_id(0)
        off = pl.multiple_of(i * B, 128)
        h = hist_ref[0, pl.ds(off, B)] + hist_ref[1, pl.ds(off, B)]
        dis = lax.rsqrt(1.0 + h)
        s = a0_ref[...] + a1_ref[...] - y_ref[...]
        o_ref[...] = jnp.maximum(s * dis[:, None] + b_ref[...], 0.0)

    return pl.pallas_call(
        body,
        grid=(pl.cdiv(N, B),),
        in_specs=[
            pl.BlockSpec((NC, NP), lambda i: (0, 0)),
            pl.BlockSpec((B, D), lambda i: (i, 0)),
            pl.BlockSpec((B, D), lambda i: (i, 0)),
            pl.BlockSpec((B, D), lambda i: (i, 0)),
            pl.BlockSpec((1, D), lambda i: (0, 0)),
        ],
        out_specs=pl.BlockSpec((B, D), lambda i: (i, 0)),
        out_shape=jax.ShapeDtypeStruct((N, D), jnp.float32),
    )


def kernel(x, edge_index, W, b):
    N, D = x.shape
    E = edge_index.shape[1]
    NP = 10240
    ei = edge_index.astype(jnp.int32)
    src3 = ei[0].reshape(-1, WIN, CS)
    dst3s = ei[1].reshape(-1, WIN, CS)
    dst3h = ei[1].reshape(NW, -1, CH)

    hist = _hist_call(NP, E)(dst3h)
    xw = _matmul_call(N, D)(x, W)
    y = _scale_call(N, NP, D)(hist, xw)
    acc0, acc1 = _scatter_call(N, E, D)(y, src3, dst3s)
    out = _final_call(N, NP, D)(hist, acc0, acc1, y, b.reshape(1, D))
    return out


# speedup vs baseline: 1.0540x; 1.0540x over previous
import functools

import jax
import jax.numpy as jnp
from jax import lax
from jax.experimental import pallas as pl
from jax.experimental.pallas import tpu as pltpu
from jax.experimental.pallas import tpu_sc as plsc

NC, NS = 2, 16
NW = NC * NS
CH = 80
HQ = 8
CS = 125
WIN = 16
B = 1024


def _hist_call(NP, E):
    EPW = E // NW
    NCH = EPW // CH
    mesh = plsc.VectorSubcoreMesh(core_axis_name="c", subcore_axis_name="s")

    ZB = NP // NS

    @functools.partial(
        pl.kernel,
        out_type=jax.ShapeDtypeStruct((NC, NP), jnp.float32),
        mesh=mesh,
        scratch_types=[
            pltpu.VMEM((NCH, CH), jnp.int32),
            pltpu.VMEM((CH,), jnp.float32),
            pltpu.VMEM((ZB,), jnp.float32),
            pltpu.VMEM_SHARED((NP,), jnp.float32),
            pltpu.SemaphoreType.DMA,
        ],
    )
    def hist(dst3, out, idx_v, ones_v, zb, hist_sh, sem):
        cid = lax.axis_index("c")
        sid = lax.axis_index("s")
        wid = sid * NC + cid
        pltpu.sync_copy(dst3.at[wid], idx_v)
        for k in range(CH // 16):
            ones_v[pl.ds(k * 16, 16)] = jnp.ones((16,), jnp.float32)
        for k in range(ZB // 16):
            zb[pl.ds(k * 16, 16)] = jnp.zeros((16,), jnp.float32)
        pltpu.sync_copy(zb, hist_sh.at[pl.ds(sid * ZB, ZB)])
        plsc.subcore_barrier()

        def fire(j):
            pltpu.async_copy(ones_v, hist_sh.at[idx_v.at[j]], sem, add=True)

        def drain():
            pltpu.make_async_copy(ones_v, hist_sh.at[idx_v.at[0]], sem).wait()

        for j in range(HQ):
            fire(j)

        def body(j, carry):
            drain()
            fire(j + HQ)
            return carry

        lax.fori_loop(0, NCH - HQ, body, 0)
        for _ in range(HQ):
            drain()
        plsc.subcore_barrier()

        @pl.when(sid == 0)
        def _():
            pltpu.sync_copy(hist_sh, out.at[cid])

    return hist


def _scatter_call(N, E, D):
    EPW = E // NW
    NCH = EPW // CS
    NWIN = NCH // WIN
    RPT = 632
    RLAST = N - (NS - 1) * RPT
    mesh = plsc.VectorSubcoreMesh(core_axis_name="c", subcore_axis_name="s")

    @functools.partial(
        pl.kernel,
        out_type=[
            jax.ShapeDtypeStruct((N, D), jnp.float32),
            jax.ShapeDtypeStruct((N, D), jnp.float32),
        ],
        mesh=mesh,
        scratch_types=[
            pltpu.VMEM((WIN, CS), jnp.int32),
            pltpu.VMEM((WIN, CS), jnp.int32),
            pltpu.VMEM((WIN, CS), jnp.int32),
            pltpu.VMEM((WIN, CS), jnp.int32),
            pltpu.VMEM((2, CS, D), jnp.float32),
            pltpu.VMEM_SHARED((N, D), jnp.float32),
            pltpu.SemaphoreType.DMA,
            pltpu.SemaphoreType.DMA,
            pltpu.SemaphoreType.DMA,
            pltpu.SemaphoreType.DMA,
        ],
    )
    def scat(y, src4, dst4, out0, out1,
             iws0, iws1, iwd0, iwd1, rows, acc, g0, g1, i0, i1):
        cid = lax.axis_index("c")
        sid = lax.axis_index("s")
        wid = sid * NC + cid
        wbase = wid * NWIN
        r0 = pl.multiple_of(sid * RPT, 8)

        wbuf = ((iws0, iwd0), (iws1, iwd1))
        isem = (i0, i1)
        gsem = (g0, g1)

        def start_window(w, p):
            pltpu.async_copy(src4.at[wbase + w], wbuf[p][0], isem[p])
            pltpu.async_copy(dst4.at[wbase + w], wbuf[p][1], isem[p])

        def wait_window(p):
            pltpu.make_async_copy(src4.at[0], wbuf[p][0], isem[p]).wait()
            pltpu.make_async_copy(src4.at[0], wbuf[p][1], isem[p]).wait()

        def start_gather(idx_row, buf):
            pltpu.async_copy(y.at[idx_row], rows.at[buf], gsem[buf])

        def wait_gather(buf):
            pltpu.make_async_copy(y.at[iws0.at[0]], rows.at[buf], gsem[buf]).wait()

        def scatter(idx_row, buf):
            pltpu.sync_copy(rows.at[buf], acc.at[idx_row], add=True)

        start_window(0, 0)
        start_window(1, 1)
        wait_window(0)
        start_gather(iws0.at[0], 0)
        start_gather(iws0.at[1], 1)

        @pl.when(sid < NS - 1)
        def _():
            pltpu.sync_copy(y.at[pl.ds(r0, RPT)], acc.at[pl.ds(r0, RPT)])

        @pl.when(sid == NS - 1)
        def _():
            pltpu.sync_copy(y.at[pl.ds(r0, RLAST)], acc.at[pl.ds(r0, RLAST)])

        plsc.subcore_barrier()

        for w in range(NWIN):
            p = w % 2
            iws, iwd = wbuf[p]

            def pair(k2, carry, iws=iws, iwd=iwd):
                k = 2 * k2
                wait_gather(0)
                scatter(iwd.at[k], 0)
                start_gather(iws.at[k + 2], 0)
                wait_gather(1)
                scatter(iwd.at[k + 1], 1)
                start_gather(iws.at[k + 3], 1)
                return carry

            lax.fori_loop(0, (WIN - 2) // 2, pair, 0)
            wait_gather(0)
            scatter(iwd.at[WIN - 2], 0)
            if w + 1 < NWIN:
                wait_window(1 - p)
                start_gather(wbuf[1 - p][0].at[0], 0)
            wait_gather(1)
            scatter(iwd.at[WIN - 1], 1)
            if w + 1 < NWIN:
                start_gather(wbuf[1 - p][0].at[1], 1)
                if w + 2 < NWIN:
                    start_window(w + 2, p)

        plsc.subcore_barrier()

        def copy_out(dst_ref):
            @pl.when(sid < NS - 1)
            def _():
                pltpu.sync_copy(acc.at[pl.ds(r0, RPT)], dst_ref.at[pl.ds(r0, RPT)])

            @pl.when(sid == NS - 1)
            def _():
                pltpu.sync_copy(acc.at[pl.ds(r0, RLAST)], dst_ref.at[pl.ds(r0, RLAST)])

        @pl.when(cid == 0)
        def _():
            copy_out(out0)

        @pl.when(cid == 1)
        def _():
            copy_out(out1)

    return scat


def _prep_call(N, NP, D):
    def body(hist_ref, x_ref, w_ref, y_ref):
        i = pl.program_id(0)
        off = pl.multiple_of(i * B, 128)
        h = hist_ref[0, pl.ds(off, B)] + hist_ref[1, pl.ds(off, B)]
        dis = lax.rsqrt(1.0 + h)
        xw = jnp.dot(x_ref[...], w_ref[...], preferred_element_type=jnp.float32)
        y_ref[...] = xw * dis[:, None]

    return pl.pallas_call(
        body,
        grid=(pl.cdiv(N, B),),
        in_specs=[
            pl.BlockSpec((NC, NP), lambda i: (0, 0)),
            pl.BlockSpec((B, D), lambda i: (i, 0)),
            pl.BlockSpec((D, D), lambda i: (0, 0)),
        ],
        out_specs=pl.BlockSpec((B, D), lambda i: (i, 0)),
        out_shape=jax.ShapeDtypeStruct((N, D), jnp.float32),
    )


def _final_call(N, NP, D):
    def body(hist_ref, a0_ref, a1_ref, y_ref, b_ref, o_ref):
        i = pl.program_id(0)
        off = pl.multiple_of(i * B, 128)
        h = hist_ref[0, pl.ds(off, B)] + hist_ref[1, pl.ds(off, B)]
        dis = lax.rsqrt(1.0 + h)
        s = a0_ref[...] + a1_ref[...] - y_ref[...]
        o_ref[...] = jnp.maximum(s * dis[:, None] + b_ref[...], 0.0)

    return pl.pallas_call(
        body,
        grid=(pl.cdiv(N, B),),
        in_specs=[
            pl.BlockSpec((NC, NP), lambda i: (0, 0)),
            pl.BlockSpec((B, D), lambda i: (i, 0)),
            pl.BlockSpec((B, D), lambda i: (i, 0)),
            pl.BlockSpec((B, D), lambda i: (i, 0)),
            pl.BlockSpec((1, D), lambda i: (0, 0)),
        ],
        out_specs=pl.BlockSpec((B, D), lambda i: (i, 0)),
        out_shape=jax.ShapeDtypeStruct((N, D), jnp.float32),
    )


def kernel(x, edge_index, W, b):
    N, D = x.shape
    E = edge_index.shape[1]
    NP = 10240
    ei = edge_index.astype(jnp.int32)
    src3 = ei[0].reshape(-1, WIN, CS)
    dst3s = ei[1].reshape(-1, WIN, CS)
    dst3h = ei[1].reshape(NW, -1, CH)

    hist = _hist_call(NP, E)(dst3h)
    y = _prep_call(N, NP, D)(hist, x, W)
    acc0, acc1 = _scatter_call(N, E, D)(y, src3, dst3s)
    out = _final_call(N, NP, D)(hist, acc0, acc1, y, b.reshape(1, D))
    return out
